# Initial kernel scaffold; baseline (speedup 1.0000x reference)
import jax
import jax.numpy as jnp
from jax.experimental import pallas as pl


def kernel(x, adj, struc_feat, idx, labels, W_heads, a_heads, W_out, a_out, enc_W, enc_b, W_ss, a_ss):
    raise NotImplementedError("write your pallas kernel here")


# speedup vs baseline: 56.1966x; 56.1966x over previous
import jax
import jax.numpy as jnp
from jax.experimental import pallas as pl

N = 10000
NH = 4
FH = 64
NC = 40
BLK = 200
NBLK = N // BLK


def _elu(o):
    return jnp.where(o > 0, o, jnp.exp(o) - 1.0)


def _prep1_body(x_ref, w_ref, a_ref, haug_ref, colsc_ref, rowsS_ref):
    xb = x_ref[...]
    cols = []
    for i in range(NH):
        h = jax.lax.dot_general(xb, w_ref[i], (((1,), (0,)), ((), ())),
                                preferred_element_type=jnp.float32)
        a1 = a_ref[i][:, :FH]
        a2 = a_ref[i][:, FH:]
        f1 = jax.lax.dot_general(h, a1, (((1,), (1,)), ((), ())),
                                 preferred_element_type=jnp.float32)
        f2 = jax.lax.dot_general(h, a2, (((1,), (1,)), ((), ())),
                                 preferred_element_type=jnp.float32)
        haug_ref[i, :, 0:FH] = h
        haug_ref[i, :, FH:FH + 1] = jnp.ones_like(f1)
        haug_ref[i, :, FH + 1:] = jnp.zeros((BLK, 7), jnp.float32)
        cols += [-f1, jnp.exp(-f1), jnp.exp(-0.2 * f1), f2]
    s = jnp.concatenate(cols, axis=1)
    colsc_ref[...] = s
    rows = []
    for i in range(NH):
        f2 = s[:, 4 * i + 3:4 * i + 4]
        rows += [f2, jnp.exp(-f2), jnp.exp(-0.2 * f2), jnp.zeros_like(f2)]
    rowsS_ref[...] = jnp.concatenate(rows, axis=1)


def _prep1(x, W_heads, a_heads):
    return pl.pallas_call(
        _prep1_body,
        grid=(NBLK,),
        in_specs=[
            pl.BlockSpec((BLK, 256), lambda i: (i, 0)),
            pl.BlockSpec((NH, 256, FH), lambda i: (0, 0, 0)),
            pl.BlockSpec((NH, 1, 2 * FH), lambda i: (0, 0, 0)),
        ],
        out_specs=[
            pl.BlockSpec((NH, BLK, 72), lambda i: (0, i, 0)),
            pl.BlockSpec((BLK, 16), lambda i: (i, 0)),
            pl.BlockSpec((BLK, 16), lambda i: (i, 0)),
        ],
        out_shape=[
            jax.ShapeDtypeStruct((NH, N, 72), jnp.float32),
            jax.ShapeDtypeStruct((N, 16), jnp.float32),
            jax.ShapeDtypeStruct((N, 16), jnp.float32),
        ],
    )(x, W_heads, a_heads)


def _pass1_body(adj_ref, colsc_ref, rowsT_ref, haug_ref, h1_ref):
    adj = adj_ref[...]
    for i in range(NH):
        nf1 = colsc_ref[:, 4 * i:4 * i + 1]
        ea1 = colsc_ref[:, 4 * i + 1:4 * i + 2]
        eb1 = colsc_ref[:, 4 * i + 2:4 * i + 3]
        f2r = rowsT_ref[4 * i:4 * i + 1, :]
        ea2 = rowsT_ref[4 * i + 1:4 * i + 2, :]
        eb2 = rowsT_ref[4 * i + 2:4 * i + 3, :]
        w = jnp.where(f2r > nf1, ea1 * ea2, eb1 * eb2) * adj
        hp = jax.lax.dot_general(w, haug_ref[i], (((1,), (0,)), ((), ())),
                                 preferred_element_type=jnp.float32)
        o = hp[:, :FH] / hp[:, FH:FH + 1]
        h1_ref[:, FH * i:FH * (i + 1)] = _elu(o)


def _pass1(adj, colsc, rowsT, haug):
    return pl.pallas_call(
        _pass1_body,
        grid=(NBLK,),
        in_specs=[
            pl.BlockSpec((BLK, N), lambda i: (i, 0)),
            pl.BlockSpec((BLK, 16), lambda i: (i, 0)),
            pl.BlockSpec((16, N), lambda i: (0, 0)),
            pl.BlockSpec((NH, N, 72), lambda i: (0, 0, 0)),
        ],
        out_specs=pl.BlockSpec((BLK, 256), lambda i: (i, 0)),
        out_shape=jax.ShapeDtypeStruct((N, 256), jnp.float32),
    )(adj, colsc, rowsT, haug)


def _prep2_body(h1_ref, struc_ref, wout_ref, awout_ref, encwt_ref, encb_ref,
                wss_ref, ass_ref, gaug_ref, colsc2_ref, rows2S_ref):
    h1 = h1_ref[...]
    g = jax.lax.dot_general(h1, wout_ref[...], (((1,), (0,)), ((), ())),
                            preferred_element_type=jnp.float32)
    a1o = awout_ref[:, :NC]
    a2o = awout_ref[:, NC:]
    u2 = jax.lax.dot_general(g, a1o, (((1,), (1,)), ((), ())),
                             preferred_element_type=jnp.float32)
    v2 = jax.lax.dot_general(g, a2o, (((1,), (1,)), ((), ())),
                             preferred_element_type=jnp.float32)
    y = jax.lax.dot_general(h1, encwt_ref[...], (((1,), (0,)), ((), ())),
                            preferred_element_type=jnp.float32) + encb_ref[...]
    p = jax.lax.dot_general(y, wss_ref[...], (((1,), (0,)), ((), ())),
                            preferred_element_type=jnp.float32)
    q = jax.lax.dot_general(struc_ref[...], wss_ref[...], (((1,), (0,)), ((), ())),
                            preferred_element_type=jnp.float32)
    a0 = ass_ref[0, 0]
    a1s = ass_ref[0, 1]
    gaug_ref[:, 0:NC] = g
    gaug_ref[:, NC:NC + 1] = jnp.ones_like(p)
    gaug_ref[:, NC + 1:NC + 2] = p
    gaug_ref[:, NC + 2:NC + 3] = q
    gaug_ref[:, NC + 3:] = jnp.zeros((BLK, 5), jnp.float32)
    f1s = [u2, a0 * p, a0 * q]
    f2s = [v2, a1s * p, a1s * q]
    cols, rows = [], []
    for f1, f2 in zip(f1s, f2s):
        cols += [-f1, jnp.exp(-f1), jnp.exp(-0.2 * f1), jnp.zeros_like(f1)]
        rows += [f2, jnp.exp(-f2), jnp.exp(-0.2 * f2), jnp.zeros_like(f2)]
    colsc2_ref[...] = jnp.concatenate(cols, axis=1)
    rows2S_ref[...] = jnp.concatenate(rows, axis=1)


def _prep2(h1, struc, W_out, a_out, enc_WT, enc_b, W_ss, a_ss):
    return pl.pallas_call(
        _prep2_body,
        grid=(NBLK,),
        in_specs=[
            pl.BlockSpec((BLK, 256), lambda i: (i, 0)),
            pl.BlockSpec((BLK, 64), lambda i: (i, 0)),
            pl.BlockSpec((256, NC), lambda i: (0, 0)),
            pl.BlockSpec((1, 2 * NC), lambda i: (0, 0)),
            pl.BlockSpec((256, 64), lambda i: (0, 0)),
            pl.BlockSpec((1, 64), lambda i: (0, 0)),
            pl.BlockSpec((64, 1), lambda i: (0, 0)),
            pl.BlockSpec((1, 2), lambda i: (0, 0)),
        ],
        out_specs=[
            pl.BlockSpec((BLK, 48), lambda i: (i, 0)),
            pl.BlockSpec((BLK, 12), lambda i: (i, 0)),
            pl.BlockSpec((BLK, 12), lambda i: (i, 0)),
        ],
        out_shape=[
            jax.ShapeDtypeStruct((N, 48), jnp.float32),
            jax.ShapeDtypeStruct((N, 12), jnp.float32),
            jax.ShapeDtypeStruct((N, 12), jnp.float32),
        ],
    )(h1, struc, W_out, a_out, enc_WT, enc_b, W_ss, a_ss)


def _pass2_body(adj_ref, colsc_ref, rowsT_ref, gaug_ref, lab_ref,
                x2_ref, pllab_ref, y2_ref, sf_ref):
    adj = adj_ref[...]
    hps = []
    for li in range(3):
        nf1 = colsc_ref[:, 4 * li:4 * li + 1]
        ea1 = colsc_ref[:, 4 * li + 1:4 * li + 2]
        eb1 = colsc_ref[:, 4 * li + 2:4 * li + 3]
        f2r = rowsT_ref[4 * li:4 * li + 1, :]
        ea2 = rowsT_ref[4 * li + 1:4 * li + 2, :]
        eb2 = rowsT_ref[4 * li + 2:4 * li + 3, :]
        w = jnp.where(f2r > nf1, ea1 * ea2, eb1 * eb2) * adj
        hps.append(jax.lax.dot_general(w, gaug_ref[...], (((1,), (0,)), ((), ())),
                                       preferred_element_type=jnp.float32))
    x2 = hps[0][:, :NC] / hps[0][:, NC:NC + 1]
    x2 = _elu(x2)
    x2_ref[...] = x2
    y2_ref[...] = _elu(hps[1][:, NC + 1:NC + 2] / hps[1][:, NC:NC + 1])
    sf_ref[...] = _elu(hps[2][:, NC + 2:NC + 3] / hps[2][:, NC:NC + 1])
    m = jnp.max(x2, axis=1, keepdims=True)
    lse = jnp.log(jnp.sum(jnp.exp(x2 - m), axis=1, keepdims=True)) + m
    cls = jax.lax.broadcasted_iota(jnp.int32, (BLK, NC), 1).astype(jnp.float32)
    onehot = (cls == lab_ref[...]).astype(jnp.float32)
    pllab_ref[...] = jnp.sum(onehot * (x2 - lse), axis=1, keepdims=True)


def _pass2(adj, colsc2, rows2T, gaug, labf):
    return pl.pallas_call(
        _pass2_body,
        grid=(NBLK,),
        in_specs=[
            pl.BlockSpec((BLK, N), lambda i: (i, 0)),
            pl.BlockSpec((BLK, 12), lambda i: (i, 0)),
            pl.BlockSpec((12, N), lambda i: (0, 0)),
            pl.BlockSpec((N, 48), lambda i: (0, 0)),
            pl.BlockSpec((BLK, 1), lambda i: (i, 0)),
        ],
        out_specs=[
            pl.BlockSpec((BLK, NC), lambda i: (i, 0)),
            pl.BlockSpec((BLK, 1), lambda i: (i, 0)),
            pl.BlockSpec((BLK, 1), lambda i: (i, 0)),
            pl.BlockSpec((BLK, 1), lambda i: (i, 0)),
        ],
        out_shape=[
            jax.ShapeDtypeStruct((N, NC), jnp.float32),
            jax.ShapeDtypeStruct((N, 1), jnp.float32),
            jax.ShapeDtypeStruct((N, 1), jnp.float32),
            jax.ShapeDtypeStruct((N, 1), jnp.float32),
        ],
    )(adj, colsc2, rows2T, gaug, labf)


def _fin_body(idx_ref, pllab_ref, y2_ref, sf_ref, nll_ref, z_ref):
    pid = pl.program_id(0)
    rows = jax.lax.broadcasted_iota(jnp.int32, (N, 128), 0).astype(jnp.float32)
    hit = (rows == idx_ref[...]).astype(jnp.float32)
    part = jnp.sum(hit * pllab_ref[...], keepdims=True)

    @pl.when(pid == 0)
    def _():
        nll_ref[...] = jnp.zeros((1, 1), jnp.float32)
        d = y2_ref[...] - sf_ref[...]
        z_ref[...] = jnp.sum(d * d, keepdims=True)

    nll_ref[...] = nll_ref[...] + part


def _finalize(idxf, pllab, y2, sf):
    return pl.pallas_call(
        _fin_body,
        grid=(8,),
        in_specs=[
            pl.BlockSpec((1, 128), lambda i: (0, i)),
            pl.BlockSpec((N, 1), lambda i: (0, 0)),
            pl.BlockSpec((N, 1), lambda i: (0, 0)),
            pl.BlockSpec((N, 1), lambda i: (0, 0)),
        ],
        out_specs=[
            pl.BlockSpec((1, 1), lambda i: (0, 0)),
            pl.BlockSpec((1, 1), lambda i: (0, 0)),
        ],
        out_shape=[
            jax.ShapeDtypeStruct((1, 1), jnp.float32),
            jax.ShapeDtypeStruct((1, 1), jnp.float32),
        ],
    )(idxf, pllab, y2, sf)


def kernel(x, adj, struc_feat, idx, labels, W_heads, a_heads, W_out, a_out,
           enc_W, enc_b, W_ss, a_ss):
    haug, colsc, rowsS = _prep1(x, W_heads, a_heads)
    h1 = _pass1(adj, colsc, rowsS.T, haug)
    enc_WT = enc_W.T
    enc_b2 = enc_b.reshape(1, 64)
    gaug, colsc2, rows2S = _prep2(h1, struc_feat, W_out, a_out, enc_WT,
                                  enc_b2, W_ss, a_ss)
    labf = labels.astype(jnp.float32).reshape(N, 1)
    x2, pllab, y2, sf = _pass2(adj, colsc2, rows2S.T, gaug, labf)
    idxf = jnp.concatenate(
        [idx.astype(jnp.float32), jnp.full((24,), -1.0, jnp.float32)]
    ).reshape(1, 1024)
    nllsum, zsum = _finalize(idxf, pllab, y2, sf)
    nll = -nllsum[0, 0] / idx.shape[0]
    z = zsum[0, 0] / N
    return (nll, z, x2)


# speedup vs baseline: 77.7277x; 1.3831x over previous
import functools

import jax
import jax.numpy as jnp
from jax import lax
from jax.experimental import pallas as pl
from jax.experimental.pallas import tpu as pltpu, tpu_sc as plsc

N = 10000
NPAD = 10240
TW = 128
NIDX = 1024
IPT = NIDX // 32
RPT = NPAD // 32
NH = 4
FH = 64
NC = 40
BLK = 400
NBLK = N // BLK


def _elu(o):
    return jnp.where(o > 0, o, jnp.exp(o) - 1.0)


def _prep1_body(x_ref, w_ref, a_ref, haug_ref, colsc_ref, rowsS_ref):
    xb = x_ref[...]
    cols = []
    for i in range(NH):
        h = jax.lax.dot_general(xb, w_ref[i], (((1,), (0,)), ((), ())),
                                preferred_element_type=jnp.float32)
        a1 = a_ref[i][:, :FH]
        a2 = a_ref[i][:, FH:]
        f1 = jax.lax.dot_general(h, a1, (((1,), (1,)), ((), ())),
                                 preferred_element_type=jnp.float32)
        f2 = jax.lax.dot_general(h, a2, (((1,), (1,)), ((), ())),
                                 preferred_element_type=jnp.float32)
        haug_ref[i, :, 0:FH] = h
        haug_ref[i, :, FH:FH + 1] = jnp.ones_like(f1)
        haug_ref[i, :, FH + 1:] = jnp.zeros((BLK, 7), jnp.float32)
        cols += [jnp.exp(0.8 * f1), jnp.zeros_like(f1), jnp.zeros_like(f1), f2]
    s = jnp.concatenate(cols, axis=1)
    colsc_ref[...] = s
    rows = []
    for i in range(NH):
        f2 = s[:, 4 * i + 3:4 * i + 4]
        rows += [f2, jnp.exp(-f2), jnp.exp(-0.2 * f2), jnp.zeros_like(f2)]
    rowsS_ref[...] = jnp.concatenate(rows, axis=1)


def _prep1(x, W_heads, a_heads):
    return pl.pallas_call(
        _prep1_body,
        grid=(NBLK,),
        in_specs=[
            pl.BlockSpec((BLK, 256), lambda i: (i, 0)),
            pl.BlockSpec((NH, 256, FH), lambda i: (0, 0, 0)),
            pl.BlockSpec((NH, 1, 2 * FH), lambda i: (0, 0, 0)),
        ],
        out_specs=[
            pl.BlockSpec((NH, BLK, 72), lambda i: (0, i, 0)),
            pl.BlockSpec((BLK, 16), lambda i: (i, 0)),
            pl.BlockSpec((BLK, 16), lambda i: (i, 0)),
        ],
        out_shape=[
            jax.ShapeDtypeStruct((NH, N, 72), jnp.float32),
            jax.ShapeDtypeStruct((N, 16), jnp.float32),
            jax.ShapeDtypeStruct((N, 16), jnp.float32),
        ],
    )(x, W_heads, a_heads)


def _pass1_body(adj_ref, colsc_ref, rowsT_ref, haug_ref, h1_ref):
    adj = adj_ref[...]
    for i in range(NH):
        r1 = colsc_ref[:, 4 * i:4 * i + 1]
        ea2 = rowsT_ref[4 * i + 1:4 * i + 2, :]
        eb2 = rowsT_ref[4 * i + 2:4 * i + 3, :]
        w = jnp.minimum(ea2, r1 * eb2) * adj
        hp = jax.lax.dot_general(w, haug_ref[i], (((1,), (0,)), ((), ())),
                                 preferred_element_type=jnp.float32)
        o = hp[:, :FH] / hp[:, FH:FH + 1]
        h1_ref[:, FH * i:FH * (i + 1)] = _elu(o)


def _pass1(adj, colsc, rowsT, haug):
    return pl.pallas_call(
        _pass1_body,
        grid=(NBLK,),
        in_specs=[
            pl.BlockSpec((BLK, N), lambda i: (i, 0)),
            pl.BlockSpec((BLK, 16), lambda i: (i, 0)),
            pl.BlockSpec((16, N), lambda i: (0, 0)),
            pl.BlockSpec((NH, N, 72), lambda i: (0, 0, 0)),
        ],
        out_specs=pl.BlockSpec((BLK, 256), lambda i: (i, 0)),
        out_shape=jax.ShapeDtypeStruct((N, 256), jnp.float32),
    )(adj, colsc, rowsT, haug)


def _prep2_body(h1_ref, struc_ref, wout_ref, awout_ref, encwt_ref, encb_ref,
                wss_ref, ass_ref, gaug_ref, colsc2_ref, rows2S_ref):
    h1 = h1_ref[...]
    g = jax.lax.dot_general(h1, wout_ref[...], (((1,), (0,)), ((), ())),
                            preferred_element_type=jnp.float32)
    a1o = awout_ref[:, :NC]
    a2o = awout_ref[:, NC:]
    u2 = jax.lax.dot_general(g, a1o, (((1,), (1,)), ((), ())),
                             preferred_element_type=jnp.float32)
    v2 = jax.lax.dot_general(g, a2o, (((1,), (1,)), ((), ())),
                             preferred_element_type=jnp.float32)
    y = jax.lax.dot_general(h1, encwt_ref[...], (((1,), (0,)), ((), ())),
                            preferred_element_type=jnp.float32) + encb_ref[...]
    p = jax.lax.dot_general(y, wss_ref[...], (((1,), (0,)), ((), ())),
                            preferred_element_type=jnp.float32)
    q = jax.lax.dot_general(struc_ref[...], wss_ref[...], (((1,), (0,)), ((), ())),
                            preferred_element_type=jnp.float32)
    a0 = ass_ref[0, 0]
    a1s = ass_ref[0, 1]
    gaug_ref[:, 0:NC] = g
    gaug_ref[:, NC:NC + 1] = jnp.ones_like(p)
    gaug_ref[:, NC + 1:NC + 2] = p
    gaug_ref[:, NC + 2:NC + 3] = q
    gaug_ref[:, NC + 3:] = jnp.zeros((BLK, 5), jnp.float32)
    f1s = [u2, a0 * p, a0 * q]
    f2s = [v2, a1s * p, a1s * q]
    cols, rows = [], []
    for f1, f2 in zip(f1s, f2s):
        cols += [jnp.exp(0.8 * f1), jnp.zeros_like(f1), jnp.zeros_like(f1),
                 jnp.zeros_like(f1)]
        rows += [f2, jnp.exp(-f2), jnp.exp(-0.2 * f2), jnp.zeros_like(f2)]
    colsc2_ref[...] = jnp.concatenate(cols, axis=1)
    rows2S_ref[...] = jnp.concatenate(rows, axis=1)


def _prep2(h1, struc, W_out, a_out, enc_WT, enc_b, W_ss, a_ss):
    return pl.pallas_call(
        _prep2_body,
        grid=(NBLK,),
        in_specs=[
            pl.BlockSpec((BLK, 256), lambda i: (i, 0)),
            pl.BlockSpec((BLK, 64), lambda i: (i, 0)),
            pl.BlockSpec((256, NC), lambda i: (0, 0)),
            pl.BlockSpec((1, 2 * NC), lambda i: (0, 0)),
            pl.BlockSpec((256, 64), lambda i: (0, 0)),
            pl.BlockSpec((1, 64), lambda i: (0, 0)),
            pl.BlockSpec((64, 1), lambda i: (0, 0)),
            pl.BlockSpec((1, 2), lambda i: (0, 0)),
        ],
        out_specs=[
            pl.BlockSpec((BLK, 48), lambda i: (i, 0)),
            pl.BlockSpec((BLK, 12), lambda i: (i, 0)),
            pl.BlockSpec((BLK, 12), lambda i: (i, 0)),
        ],
        out_shape=[
            jax.ShapeDtypeStruct((N, 48), jnp.float32),
            jax.ShapeDtypeStruct((N, 12), jnp.float32),
            jax.ShapeDtypeStruct((N, 12), jnp.float32),
        ],
    )(h1, struc, W_out, a_out, enc_WT, enc_b, W_ss, a_ss)


def _pass2_body(adj_ref, colsc_ref, rowsT_ref, gaug_ref, lab_ref,
                x2_ref, tab_ref):
    adj = adj_ref[...]
    hps = []
    for li in range(3):
        r1 = colsc_ref[:, 4 * li:4 * li + 1]
        ea2 = rowsT_ref[4 * li + 1:4 * li + 2, :]
        eb2 = rowsT_ref[4 * li + 2:4 * li + 3, :]
        w = jnp.minimum(ea2, r1 * eb2) * adj
        hps.append(jax.lax.dot_general(w, gaug_ref[...], (((1,), (0,)), ((), ())),
                                       preferred_element_type=jnp.float32))
    x2 = hps[0][:, :NC] / hps[0][:, NC:NC + 1]
    x2 = _elu(x2)
    x2_ref[...] = x2
    y2 = _elu(hps[1][:, NC + 1:NC + 2] / hps[1][:, NC:NC + 1])
    sf = _elu(hps[2][:, NC + 2:NC + 3] / hps[2][:, NC:NC + 1])
    m = jnp.max(x2, axis=1, keepdims=True)
    lse = jnp.log(jnp.sum(jnp.exp(x2 - m), axis=1, keepdims=True)) + m
    cls = jax.lax.broadcasted_iota(jnp.int32, (BLK, NC), 1).astype(jnp.float32)
    onehot = (cls == lab_ref[...]).astype(jnp.float32)
    pllab = jnp.sum(onehot * (x2 - lse), axis=1, keepdims=True)
    tab_ref[:, 0:1] = pllab
    tab_ref[:, 1:2] = y2 - sf
    tab_ref[:, 2:] = jnp.zeros((BLK, TW - 2), jnp.float32)


def _pass2(adj, colsc2, rows2T, gaug, labf):
    return pl.pallas_call(
        _pass2_body,
        grid=(NBLK,),
        in_specs=[
            pl.BlockSpec((BLK, N), lambda i: (i, 0)),
            pl.BlockSpec((BLK, 12), lambda i: (i, 0)),
            pl.BlockSpec((12, N), lambda i: (0, 0)),
            pl.BlockSpec((N, 48), lambda i: (0, 0)),
            pl.BlockSpec((BLK, 1), lambda i: (i, 0)),
        ],
        out_specs=[
            pl.BlockSpec((BLK, NC), lambda i: (i, 0)),
            pl.BlockSpec((BLK, TW), lambda i: (i, 0)),
        ],
        out_shape=[
            jax.ShapeDtypeStruct((N, NC), jnp.float32),
            jax.ShapeDtypeStruct((NPAD, TW), jnp.float32),
        ],
    )(adj, colsc2, rows2T, gaug, labf)


def _fin_sc_body(tab_hbm, idx_hbm, msk_hbm, part_hbm, idxbuf, rows_v, zbuf,
                 outbuf, mskbuf, sem):
    wid = lax.axis_index("s") * 2 + lax.axis_index("c")
    zero16 = jnp.zeros((16,), jnp.float32)
    pltpu.sync_copy(msk_hbm, mskbuf)
    m0 = mskbuf[0, pl.ds(0, 16)]
    m1 = mskbuf[1, pl.ds(0, 16)]

    ibase = wid * IPT
    pltpu.sync_copy(idx_hbm.at[pl.ds(ibase, IPT)], idxbuf)
    pltpu.async_copy(tab_hbm.at[idxbuf], rows_v, sem).wait()
    acc_nll = zero16
    for r in range(IPT):
        mf = jnp.where((ibase + r) < 1000, 1.0, 0.0).astype(jnp.float32)
        acc_nll = acc_nll + rows_v[r, pl.ds(0, 16)] * (m0 * mf)

    rbase = wid * RPT
    acc_z = zero16
    pltpu.sync_copy(tab_hbm.at[pl.ds(rbase, RPT)], zbuf)
    for r in range(RPT):
        mf = jnp.where((rbase + r) < N, 1.0, 0.0).astype(jnp.float32)
        d = zbuf[r, pl.ds(0, 16)] * (m1 * mf)
        acc_z = acc_z + d * d

    for r in range(8):
        for k in range(8):
            outbuf[r, pl.ds(16 * k, 16)] = zero16
    outbuf[0, pl.ds(0, 16)] = acc_nll
    outbuf[1, pl.ds(0, 16)] = acc_z
    pltpu.sync_copy(outbuf, part_hbm.at[wid])


_fin_sc = functools.partial(
    pl.kernel,
    out_type=jax.ShapeDtypeStruct((32, 8, TW), jnp.float32),
    mesh=plsc.VectorSubcoreMesh(core_axis_name="c", subcore_axis_name="s"),
    scratch_types=[
        pltpu.VMEM((IPT,), jnp.int32),
        pltpu.VMEM((IPT, TW), jnp.float32),
        pltpu.VMEM((RPT, TW), jnp.float32),
        pltpu.VMEM((8, TW), jnp.float32),
        pltpu.VMEM((8, TW), jnp.float32),
        pltpu.SemaphoreType.DMA,
    ],
)(_fin_sc_body)


def _fintc_body(part_ref, nll_ref, z_ref):
    nll_ref[...] = jnp.sum(part_ref[:, 0, :], keepdims=True)
    z_ref[...] = jnp.sum(part_ref[:, 1, :], keepdims=True)


def _fin_tc(part):
    return pl.pallas_call(
        _fintc_body,
        grid=(1,),
        in_specs=[pl.BlockSpec((32, 8, TW), lambda i: (0, 0, 0))],
        out_specs=[
            pl.BlockSpec((1, 1), lambda i: (0, 0)),
            pl.BlockSpec((1, 1), lambda i: (0, 0)),
        ],
        out_shape=[
            jax.ShapeDtypeStruct((1, 1), jnp.float32),
            jax.ShapeDtypeStruct((1, 1), jnp.float32),
        ],
    )(part)


def kernel(x, adj, struc_feat, idx, labels, W_heads, a_heads, W_out, a_out,
           enc_W, enc_b, W_ss, a_ss):
    haug, colsc, rowsS = _prep1(x, W_heads, a_heads)
    h1 = _pass1(adj, colsc, rowsS.T, haug)
    enc_WT = enc_W.T
    enc_b2 = enc_b.reshape(1, 64)
    gaug, colsc2, rows2S = _prep2(h1, struc_feat, W_out, a_out, enc_WT,
                                  enc_b2, W_ss, a_ss)
    labf = labels.astype(jnp.float32).reshape(N, 1)
    x2, tab = _pass2(adj, colsc2, rows2S.T, gaug, labf)
    idxp = jnp.concatenate([idx.astype(jnp.int32),
                            jnp.zeros((NIDX - idx.shape[0],), jnp.int32)])
    masks = jnp.zeros((8, TW), jnp.float32).at[0, 0].set(1.0).at[1, 1].set(1.0)
    part = _fin_sc(tab, idxp, masks)
    nllsum, zsum = _fin_tc(part)
    nll = -nllsum[0, 0] / idx.shape[0]
    z = zsum[0, 0] / N
    return (nll, z, x2)
